# Initial kernel scaffold; baseline (speedup 1.0000x reference)
import jax
import jax.numpy as jnp
from jax.experimental import pallas as pl


def kernel(input_ids, og_table, new_table):
    raise NotImplementedError("write your pallas kernel here")


# speedup vs baseline: 4.2429x; 4.2429x over previous
import functools

import jax
import jax.numpy as jnp
from jax import lax
from jax.experimental import pallas as pl
from jax.experimental.pallas import tpu as pltpu
from jax.experimental.pallas import tpu_sc as plsc

_LANES = 16
_IDXW = 128
_KIDX = 8
_KG = 4


def _sc_workers():
    try:
        info = plsc.get_sparse_core_info()
        return info.num_cores, info.num_subcores
    except Exception:
        return 2, 16


@jax.jit
def _sc_lookup(ids, og_table, new_table):
    (T,) = ids.shape
    V, D = og_table.shape
    N, _ = new_table.shape
    NC, NS = _sc_workers()
    NW = NC * NS
    ids_per_step = _KIDX * _IDXW
    half = _KG * _IDXW
    steps = T // (NW * ids_per_step)
    assert T == steps * NW * ids_per_step

    mesh = plsc.VectorSubcoreMesh(core_axis_name="c", subcore_axis_name="s",
                                  num_cores=NC, num_subcores=NS)

    @functools.partial(
        pl.kernel,
        out_type=jax.ShapeDtypeStruct((T, D), jnp.float32),
        mesh=mesh,
        compiler_params=pltpu.CompilerParams(use_tc_tiling_on_sc=False),
        scratch_types=[
            pltpu.VMEM((ids_per_step,), jnp.int32),
            pltpu.VMEM((_KIDX, _IDXW), jnp.int32),
            pltpu.VMEM((half, D), jnp.float32),
            pltpu.SemaphoreType.DMA,
            pltpu.SemaphoreType.DMA,
        ],
    )
    def k(ids_hbm, og_hbm, new_hbm, out_hbm, idx_raw, idx_safe, rows,
          gsem, csem):
        wid = lax.axis_index("s") * NC + lax.axis_index("c")

        dn = lax.GatherDimensionNumbers(
            offset_dims=(), collapsed_slice_dims=(0,), start_index_map=(0,))

        def lane_max(x):
            for sh in (1, 2, 4, 8):
                perm = lax.iota(jnp.int32, _LANES) ^ sh
                x = jnp.maximum(
                    x, lax.gather(x, perm[:, None], dn, (1,),
                                  mode=lax.GatherScatterMode.PROMISE_IN_BOUNDS))
            return x[0]

        fix_groups = half // _LANES

        def step(s, _):
            base = (wid * steps + s) * ids_per_step
            pltpu.async_copy(ids_hbm.at[pl.ds(base, ids_per_step)], idx_raw,
                             csem).wait()
            for g in range(ids_per_step // _LANES):
                kk, off = divmod(g * _LANES, _IDXW)
                v = idx_raw[pl.ds(g * _LANES, _LANES)]
                idx_safe[kk, pl.ds(off, _LANES)] = jnp.where(v >= V, v - V, v)

            def do_half(h):
                handles = [
                    pltpu.async_copy(og_hbm.at[idx_safe.at[h * _KG + kk]],
                                     rows.at[pl.ds(kk * _IDXW, _IDXW)], gsem)
                    for kk in range(_KG)
                ]
                for hd in handles:
                    hd.wait()

                def fix(g, _):
                    v = idx_raw[pl.ds((h * fix_groups + g) * _LANES, _LANES)]
                    vmax = lane_max(v.astype(jnp.float32))

                    @pl.when(vmax >= float(V))
                    def _():
                        for j in range(_LANES):
                            vj = v[j]

                            @pl.when(vj >= V)
                            def _():
                                pltpu.async_copy(
                                    new_hbm.at[pl.ds(vj - V, 1)],
                                    rows.at[pl.ds(g * _LANES + j, 1)],
                                    csem).wait()
                    return 0

                lax.fori_loop(0, fix_groups, fix, 0)
                pltpu.async_copy(rows, out_hbm.at[pl.ds(base + h * half, half)],
                                 csem).wait()

            do_half(0)
            do_half(1)
            return 0

        lax.fori_loop(0, steps, step, 0)

    return k(ids, og_table, new_table)


def kernel(input_ids, og_table, new_table):
    B, H = input_ids.shape
    V, D = og_table.shape
    ids = input_ids.reshape(B * H).astype(jnp.int32)
    out = _sc_lookup(ids, og_table, new_table)
    return out.reshape(B, H, D)


# speedup vs baseline: 4.5786x; 1.0791x over previous
import functools

import jax
import jax.numpy as jnp
from jax import lax
from jax.experimental import pallas as pl
from jax.experimental.pallas import tpu as pltpu
from jax.experimental.pallas import tpu_sc as plsc

_LANES = 16
_IDXW = 128
_KG = 4
_CH = _KG * _IDXW


def _sc_workers():
    try:
        info = plsc.get_sparse_core_info()
        return info.num_cores, info.num_subcores
    except Exception:
        return 2, 16


@jax.jit
def _sc_lookup(ids, og_table, new_table):
    (T,) = ids.shape
    V, D = og_table.shape
    N, _ = new_table.shape
    NC, NS = _sc_workers()
    NW = NC * NS
    nchunks = T // (NW * _CH)
    assert T == nchunks * NW * _CH and nchunks % 2 == 0 and nchunks >= 4

    mesh = plsc.VectorSubcoreMesh(core_axis_name="c", subcore_axis_name="s",
                                  num_cores=NC, num_subcores=NS)

    @functools.partial(
        pl.kernel,
        out_type=jax.ShapeDtypeStruct((T, D), jnp.float32),
        mesh=mesh,
        compiler_params=pltpu.CompilerParams(use_tc_tiling_on_sc=False),
        scratch_types=[
            pltpu.VMEM((_CH,), jnp.int32),
            pltpu.VMEM((_CH,), jnp.int32),
            pltpu.VMEM((_KG, _IDXW), jnp.int32),
            pltpu.VMEM((_KG, _IDXW), jnp.int32),
            pltpu.VMEM((_CH, D), jnp.float32),
            pltpu.VMEM((_CH, D), jnp.float32),
            pltpu.SemaphoreType.DMA,
            pltpu.SemaphoreType.DMA,
            pltpu.SemaphoreType.DMA,
            pltpu.SemaphoreType.DMA,
            pltpu.SemaphoreType.DMA,
            pltpu.SemaphoreType.DMA,
            pltpu.SemaphoreType.DMA,
        ],
    )
    def k(ids_hbm, og_hbm, new_hbm, out_hbm, ir0, ir1, is0, is1, rw0, rw1,
          isem0, isem1, gsem0, gsem1, osem0, osem1, fsem):
        wid = lax.axis_index("s") * NC + lax.axis_index("c")
        cbase = wid * nchunks
        IR, IS, RW = (ir0, ir1), (is0, is1), (rw0, rw1)
        ISEM, GSEM, OSEM = (isem0, isem1), (gsem0, gsem1), (osem0, osem1)

        dn = lax.GatherDimensionNumbers(
            offset_dims=(), collapsed_slice_dims=(0,), start_index_map=(0,))

        def lane_max(x):
            for sh in (1, 2, 4, 8):
                perm = lax.iota(jnp.int32, _LANES) ^ sh
                x = jnp.maximum(
                    x, lax.gather(x, perm[:, None], dn, (1,),
                                  mode=lax.GatherScatterMode.PROMISE_IN_BOUNDS))
            return x[0]

        def fetch_ids(c, b):
            pltpu.async_copy(ids_hbm.at[pl.ds((cbase + c) * _CH, _CH)],
                             IR[b], ISEM[b])

        def wait_ids(b):
            pltpu.make_async_copy(ids_hbm.at[pl.ds(0, _CH)], IR[b],
                                  ISEM[b]).wait()

        def remap(b):
            for g in range(_CH // _LANES):
                kk, off = divmod(g * _LANES, _IDXW)
                v = IR[b][pl.ds(g * _LANES, _LANES)]
                IS[b][kk, pl.ds(off, _LANES)] = jnp.where(v >= V, v - V, v)

        def fire_gathers(b):
            for kk in range(_KG):
                pltpu.async_copy(og_hbm.at[IS[b].at[kk]],
                                 RW[b].at[pl.ds(kk * _IDXW, _IDXW)], GSEM[b])

        def wait_gathers(b):
            for kk in range(_KG):
                pltpu.make_async_copy(og_hbm.at[IS[b].at[kk]],
                                      RW[b].at[pl.ds(kk * _IDXW, _IDXW)],
                                      GSEM[b]).wait()

        def fire_out(c, b):
            pltpu.async_copy(RW[b], out_hbm.at[pl.ds((cbase + c) * _CH, _CH)],
                             OSEM[b])

        def wait_out(b):
            pltpu.make_async_copy(RW[b], out_hbm.at[pl.ds(0, _CH)],
                                  OSEM[b]).wait()

        def fixup(b):
            def fix(g, _):
                v = IR[b][pl.ds(g * _LANES, _LANES)]
                vmax = lane_max(v.astype(jnp.float32))

                @pl.when(vmax >= float(V))
                def _():
                    for j in range(_LANES):
                        vj = v[j]

                        @pl.when(vj >= V)
                        def _():
                            pltpu.async_copy(
                                new_hbm.at[pl.ds(vj - V, 1)],
                                RW[b].at[pl.ds(g * _LANES + j, 1)],
                                fsem).wait()
                return 0

            lax.fori_loop(0, _CH // _LANES, fix, 0)

        fetch_ids(0, 0)
        fetch_ids(1, 1)
        wait_ids(0)
        remap(0)
        fire_gathers(0)

        def body(i, _):
            for b in (0, 1):
                c = 2 * i + b
                wait_gathers(b)
                fixup(b)
                fire_out(c, b)

                @pl.when(c + 1 < nchunks)
                def _():
                    wait_ids(b ^ 1)

                remap(b ^ 1)

                @pl.when(c + 2 < nchunks)
                def _():
                    fetch_ids(c + 2, b)

                @pl.when((c >= 1) & (c + 1 < nchunks))
                def _():
                    wait_out(b ^ 1)

                @pl.when(c + 1 < nchunks)
                def _():
                    fire_gathers(b ^ 1)
            return 0

        lax.fori_loop(0, nchunks // 2, body, 0)
        wait_out(0)
        wait_out(1)

    return k(ids, og_table, new_table)


def kernel(input_ids, og_table, new_table):
    B, H = input_ids.shape
    V, D = og_table.shape
    ids = input_ids.reshape(B * H).astype(jnp.int32)
    out = _sc_lookup(ids, og_table, new_table)
    return out.reshape(B, H, D)


# speedup vs baseline: 5.7997x; 1.2667x over previous
import functools

import jax
import jax.numpy as jnp
from jax import lax
from jax.experimental import pallas as pl
from jax.experimental.pallas import tpu as pltpu
from jax.experimental.pallas import tpu_sc as plsc

_LANES = 16
_IDXW = 128
_KG = 4
_CH = _KG * _IDXW


def _sc_workers():
    try:
        info = plsc.get_sparse_core_info()
        return info.num_cores, info.num_subcores
    except Exception:
        return 2, 16


@jax.jit
def _sc_lookup(ids, og_table, new_table):
    (T,) = ids.shape
    V, D = og_table.shape
    N, _ = new_table.shape
    NC, NS = _sc_workers()
    NW = NC * NS
    nchunks = T // (NW * _CH)
    assert T == nchunks * NW * _CH and nchunks % 2 == 0 and nchunks >= 4

    mesh = plsc.VectorSubcoreMesh(core_axis_name="c", subcore_axis_name="s",
                                  num_cores=NC, num_subcores=NS)

    @functools.partial(
        pl.kernel,
        out_type=jax.ShapeDtypeStruct((T, D), jnp.float32),
        mesh=mesh,
        compiler_params=pltpu.CompilerParams(use_tc_tiling_on_sc=False),
        scratch_types=[
            pltpu.VMEM((_CH,), jnp.int32),
            pltpu.VMEM((_CH,), jnp.int32),
            pltpu.VMEM((_KG, _IDXW), jnp.int32),
            pltpu.VMEM((_KG, _IDXW), jnp.int32),
            pltpu.VMEM((_CH, D), jnp.float32),
            pltpu.VMEM((_CH, D), jnp.float32),
            pltpu.SemaphoreType.DMA,
            pltpu.SemaphoreType.DMA,
            pltpu.SemaphoreType.DMA,
            pltpu.SemaphoreType.DMA,
            pltpu.SemaphoreType.DMA,
            pltpu.SemaphoreType.DMA,
            pltpu.SemaphoreType.DMA,
        ],
    )
    def k(ids_hbm, og_hbm, new_hbm, out_hbm, ir0, ir1, is0, is1, rw0, rw1,
          isem0, isem1, gsem0, gsem1, osem0, osem1, fsem):
        wid = lax.axis_index("s") * NC + lax.axis_index("c")
        cbase = wid * nchunks
        IR, IS, RW = (ir0, ir1), (is0, is1), (rw0, rw1)
        ISEM, GSEM, OSEM = (isem0, isem1), (gsem0, gsem1), (osem0, osem1)

        dn = lax.GatherDimensionNumbers(
            offset_dims=(), collapsed_slice_dims=(0,), start_index_map=(0,))

        def lane_max(x):
            for sh in (1, 2, 4, 8):
                perm = lax.iota(jnp.int32, _LANES) ^ sh
                x = jnp.maximum(
                    x, lax.gather(x, perm[:, None], dn, (1,),
                                  mode=lax.GatherScatterMode.PROMISE_IN_BOUNDS))
            return x[0]

        def fetch_ids(c, b):
            pltpu.async_copy(ids_hbm.at[pl.ds((cbase + c) * _CH, _CH)],
                             IR[b], ISEM[b])

        def wait_ids(b):
            pltpu.make_async_copy(ids_hbm.at[pl.ds(0, _CH)], IR[b],
                                  ISEM[b]).wait()

        def remap(b):
            for g in range(_CH // _LANES):
                kk, off = divmod(g * _LANES, _IDXW)
                v = IR[b][pl.ds(g * _LANES, _LANES)]
                IS[b][kk, pl.ds(off, _LANES)] = jnp.where(v >= V, v - V, v)

        def fire_gathers(b):
            for kk in range(_KG):
                pltpu.async_copy(og_hbm.at[IS[b].at[kk]],
                                 RW[b].at[pl.ds(kk * _IDXW, _IDXW)], GSEM[b])

        def wait_gathers(b):
            for kk in range(_KG):
                pltpu.make_async_copy(og_hbm.at[IS[b].at[kk]],
                                      RW[b].at[pl.ds(kk * _IDXW, _IDXW)],
                                      GSEM[b]).wait()

        def fire_out(c, b):
            pltpu.async_copy(RW[b], out_hbm.at[pl.ds((cbase + c) * _CH, _CH)],
                             OSEM[b])

        def wait_out(b):
            pltpu.make_async_copy(RW[b], out_hbm.at[pl.ds(0, _CH)],
                                  OSEM[b]).wait()

        def fixup(b):
            def fix(g, _):
                v = IR[b][pl.ds(g * _LANES, _LANES)]
                vmax = lane_max(v.astype(jnp.float32))

                @pl.when(vmax >= float(V))
                def _():
                    for j in range(_LANES):
                        vj = v[j]

                        @pl.when(vj >= V)
                        def _():
                            pltpu.async_copy(
                                new_hbm.at[pl.ds(vj - V, 1)],
                                RW[b].at[pl.ds(g * _LANES + j, 1)],
                                fsem).wait()
                return 0

            lax.fori_loop(0, _CH // _LANES, fix, 0)

        fetch_ids(0, 0)
        fetch_ids(1, 1)
        wait_ids(0)
        remap(0)
        fire_gathers(0)

        def body(i, _):
            for b in (0, 1):
                c = 2 * i + b
                wait_gathers(b)
                fixup(b)
                fire_out(c, b)

                @pl.when(c + 1 < nchunks)
                def _():
                    wait_ids(b ^ 1)

                remap(b ^ 1)

                @pl.when(c + 2 < nchunks)
                def _():
                    fetch_ids(c + 2, b)

                @pl.when((c >= 1) & (c + 1 < nchunks))
                def _():
                    wait_out(b ^ 1)

                @pl.when(c + 1 < nchunks)
                def _():
                    fire_gathers(b ^ 1)
            return 0

        lax.fori_loop(0, nchunks // 2, body, 0)
        wait_out(0)
        wait_out(1)

    return k(ids, og_table, new_table)


def _make_runner(B, H):
    from jax.experimental import layout as jax_layout

    lay = jax_layout.Layout(major_to_minor=(0, 1, 2), tiling=((8, 128),))
    mesh = jax.sharding.get_abstract_mesh()
    if mesh is not None and not mesh.empty:
        shard = jax.sharding.NamedSharding(mesh, jax.sharding.PartitionSpec())
    else:
        shard = jax.sharding.SingleDeviceSharding(jax.devices()[0])
    fmt = jax_layout.Format(lay, shard)

    def run(ids, og_table, new_table):
        D = og_table.shape[1]
        out = _sc_lookup(ids, og_table, new_table).reshape(B, H, D)
        return jax_layout.with_layout_constraint(out, lay)

    return run


def kernel(input_ids, og_table, new_table):
    B, H = input_ids.shape
    ids = input_ids.reshape(B * H).astype(jnp.int32)
    return _make_runner(B, H)(ids, og_table, new_table)
